# Initial kernel scaffold; baseline (speedup 1.0000x reference)
import jax
import jax.numpy as jnp
from jax.experimental import pallas as pl


def kernel(logits, targets):
    raise NotImplementedError("write your pallas kernel here")


# speedup vs baseline: 3.7067x; 3.7067x over previous
import functools

import jax
import jax.numpy as jnp
from jax import lax
from jax.experimental import pallas as pl
from jax.experimental.pallas import tpu as pltpu
from jax.experimental.pallas import tpu_sc as plsc

_NC = 2
_NS = 16
_NW = _NC * _NS
_L = 16


def _sc_gather(cls_flat, logits_flat, n, c):
    rows_w = n // _NW
    chunks = rows_w // 128
    mesh = plsc.VectorSubcoreMesh(core_axis_name="c", subcore_axis_name="s",
                                  num_cores=_NC, num_subcores=_NS)

    @functools.partial(
        pl.kernel, mesh=mesh,
        out_type=jax.ShapeDtypeStruct((n,), jnp.float32),
        scratch_types=[
            pltpu.VMEM((rows_w,), jnp.int32),
            pltpu.VMEM((rows_w,), jnp.int32),
            pltpu.VMEM((rows_w,), jnp.float32),
            pltpu.SemaphoreType.DMA,
        ],
    )
    def k(cls_hbm, logits_hbm, out_hbm, cls_v, idx_v, g_v, sem):
        wid = lax.axis_index("s") * _NC + lax.axis_index("c")
        base = wid * rows_w
        pltpu.sync_copy(cls_hbm.at[pl.ds(base, rows_w)], cls_v)

        def chunk_body(j, carry):
            for b in range(8):
                off = j * 128 + b * 16
                v = cls_v[pl.ds(off, _L)]
                row0 = base + off
                lane = lax.iota(jnp.int32, _L)
                safe = jnp.clip(v - 1, 0, c - 1)
                idx_v[pl.ds(off, _L)] = (row0 + lane) * c + safe
            pltpu.async_copy(
                logits_hbm.at[idx_v.at[pl.ds(j * 128, 128)]],
                g_v.at[pl.ds(j * 128, 128)], sem)
            return carry

        lax.fori_loop(0, chunks, chunk_body, 0)
        pltpu.make_async_copy(logits_hbm.at[pl.ds(0, rows_w)], g_v, sem).wait()
        pltpu.sync_copy(g_v, out_hbm.at[pl.ds(base, rows_w)])

    return k(cls_flat, logits_flat)


def _dense_body(x_ref, out_ref):
    i = pl.program_id(0)
    l = x_ref[:]
    th = jnp.tanh(0.5 * l)
    q = jnp.maximum(0.5 - 0.5 * th, 1e-37)
    s = jnp.sum(jnp.log(q) - q)

    @pl.when(i == 0)
    def _():
        out_ref[...] = jnp.zeros_like(out_ref)

    out_ref[...] += s.reshape(1, 1)


def _corr_body(s0_ref, g_ref, clsr_ref, iour_ref, out_ref, *, count):
    valid = clsr_ref[:] >= 1
    m = jnp.max(jnp.where(valid, iour_ref[:], 0)).astype(jnp.float32)
    lab = jnp.where(valid, iour_ref[:].astype(jnp.float32) * (1.0 / m), 0.0)
    l = g_ref[:]
    th = jnp.tanh(0.5 * l)
    q = jnp.maximum(0.5 - 0.5 * th, 1e-37)
    p = 1.0 - q
    sp = -jnp.log(q)
    c_neg = 0.75 * lab * (1.0 - sp)
    c_pos = 0.25 * lab * (sp - l - 1.0) + p - 0.75 * sp
    corr = jnp.where(p <= lab, c_pos, c_neg)
    out_ref[...] = (-0.75) * (s0_ref[...] + count) + jnp.sum(corr).reshape(1, 1)


def kernel(logits, targets):
    n, c = logits.shape
    logits_flat = logits.reshape(-1)
    cls_flat = targets[:, 0]
    clsr = cls_flat.reshape(n // 128, 128)
    iour = targets[:, 1].reshape(n // 128, 128)

    g = _sc_gather(cls_flat, logits_flat, n, c)

    rows = 1024
    wide = (n * c) // 32768
    xf = logits.reshape(32768, wide)
    s0 = pl.pallas_call(
        _dense_body,
        grid=(32768 // rows,),
        in_specs=[pl.BlockSpec((rows, wide), lambda i: (i, 0))],
        out_specs=pl.BlockSpec((1, 1), lambda i: (0, 0)),
        out_shape=jax.ShapeDtypeStruct((1, 1), jnp.float32),
    )(xf)

    out = pl.pallas_call(
        functools.partial(_corr_body, count=float(n * c)),
        in_specs=[
            pl.BlockSpec((1, 1), lambda: (0, 0)),
            pl.BlockSpec((n // 128, 128), lambda: (0, 0)),
            pl.BlockSpec((n // 128, 128), lambda: (0, 0)),
            pl.BlockSpec((n // 128, 128), lambda: (0, 0)),
        ],
        out_specs=pl.BlockSpec((1, 1), lambda: (0, 0)),
        out_shape=jax.ShapeDtypeStruct((1, 1), jnp.float32),
    )(s0, g.reshape(n // 128, 128), clsr, iour)
    return out[0, 0]
